# Initial kernel scaffold; baseline (speedup 1.0000x reference)
import jax
import jax.numpy as jnp
from jax.experimental import pallas as pl


def kernel(im1, im2, C, M1, M2):
    raise NotImplementedError("write your pallas kernel here")


# speedup vs baseline: 4.6680x; 4.6680x over previous
import functools

import numpy as np
import jax
import jax.numpy as jnp
from jax import lax
from jax.experimental import pallas as pl
from jax.experimental.pallas import tpu as pltpu
from jax.experimental.pallas import tpu_sc as plsc

IMG = 224
PIX = IMG * IMG
NB = 64
NCH = 3
NWORK = 32
TASKS = NB * NCH
TPW = TASKS // NWORK
ROWS_PER_CHUNK = 8
P = IMG * ROWS_PER_CHUNK
NCHUNK = PIX // P
GPR = IMG // 16
LO = np.float32(0.001)
HI = np.float32(IMG - 1.001)


def _frac_parts(q):
    fi = q.astype(jnp.int32)
    ff = fi.astype(jnp.float32)
    fr = q - ff
    wf = 1.0 - fr
    nz = fr != 0.0
    wc = jnp.where(nz, fr, jnp.float32(1.0))
    d = jnp.where(nz, jnp.int32(1), jnp.int32(0))
    return fi, wf, wc, d


def _sample(plane, q0, q1, mask):
    f0, wf0, wc0, d0 = _frac_parts(q0)
    f1, wf1, wc1, d1 = _frac_parts(q1)
    iff = f1 + IMG * f0
    icf = iff + IMG * d0
    ifc = iff + d1
    icc = icf + d1
    wf0m = wf0 * mask
    wc0m = wc0 * mask
    g_ff = plsc.load_gather(plane, [iff])
    g_cf = plsc.load_gather(plane, [icf])
    g_fc = plsc.load_gather(plane, [ifc])
    g_cc = plsc.load_gather(plane, [icc])
    return ((wf0m * wf1) * g_ff + (wc0m * wf1) * g_cf
            + (wf0m * wc1) * g_fc + (wc0m * wc1) * g_cc)


def _body(im1, im2, C, M1, M2, out, plane1, plane2, c0b, c1b, m1b, m2b, outb):
    wid = lax.axis_index("s") * 2 + lax.axis_index("c")
    iotaf = lax.iota(jnp.int32, 16).astype(jnp.float32)

    def do_task(t, carry):
        task = wid * TPW + t
        n = task // NCH
        ch = task - n * NCH
        pltpu.sync_copy(im1.at[n, ch], plane1)
        pltpu.sync_copy(im2.at[n, ch], plane2)

        def do_chunk(k, carry):
            off = k * P
            pltpu.sync_copy(C.at[n, 0, pl.ds(off, P)], c0b)
            pltpu.sync_copy(C.at[n, 1, pl.ds(off, P)], c1b)
            pltpu.sync_copy(M1.at[n, 0, pl.ds(off, P)], m1b)
            pltpu.sync_copy(M2.at[n, 0, pl.ds(off, P)], m2b)

            def do_row(r, carry):
                rowf = (k * ROWS_PER_CHUNK + r).astype(jnp.float32)

                def do_grp(g, carry):
                    j = (r * GPR + g) * 16
                    colf = iotaf + (g * 16).astype(jnp.float32)
                    c0 = c0b[pl.ds(j, 16)]
                    c1 = c1b[pl.ds(j, 16)]
                    qa0 = jnp.clip(rowf + c0, LO, HI)
                    qa1 = jnp.clip(colf + c1, LO, HI)
                    qb0 = jnp.clip(rowf - c0, LO, HI)
                    qb1 = jnp.clip(colf - c1, LO, HI)
                    a = _sample(plane1, qa0, qa1, m1b[pl.ds(j, 16)])
                    b = _sample(plane2, qb0, qb1, m2b[pl.ds(j, 16)])
                    outb[pl.ds(j, 16)] = a + b
                    return carry

                return lax.fori_loop(0, GPR, do_grp, carry)

            lax.fori_loop(0, ROWS_PER_CHUNK, do_row, carry)
            pltpu.sync_copy(outb, out.at[n, ch, pl.ds(off, P)])
            return carry

        return lax.fori_loop(0, NCHUNK, do_chunk, carry)

    lax.fori_loop(0, TPW, do_task, 0)


@functools.partial(jax.jit, static_argnames=())
def kernel(im1, im2, C, M1, M2):
    im1f = im1.reshape(NB, NCH, PIX)
    im2f = im2.reshape(NB, NCH, PIX)
    Cf = C.reshape(NB, 2, PIX)
    M1f = M1.reshape(NB, 1, PIX)
    M2f = M2.reshape(NB, 1, PIX)
    mesh = plsc.VectorSubcoreMesh(core_axis_name="c", subcore_axis_name="s")
    warp = pl.kernel(
        _body,
        out_type=jax.ShapeDtypeStruct((NB, NCH, PIX), jnp.float32),
        mesh=mesh,
        compiler_params=pltpu.CompilerParams(needs_layout_passes=False),
        scratch_types=[
            pltpu.VMEM((PIX,), jnp.float32),
            pltpu.VMEM((PIX,), jnp.float32),
            pltpu.VMEM((P,), jnp.float32),
            pltpu.VMEM((P,), jnp.float32),
            pltpu.VMEM((P,), jnp.float32),
            pltpu.VMEM((P,), jnp.float32),
            pltpu.VMEM((P,), jnp.float32),
        ],
    )
    out = warp(im1f, im2f, Cf, M1f, M2f)
    return out.reshape(NB, NCH, IMG, IMG)


# speedup vs baseline: 8.3775x; 1.7947x over previous
import functools

import numpy as np
import jax
import jax.numpy as jnp
from jax import lax
from jax.experimental import pallas as pl
from jax.experimental.pallas import tpu as pltpu
from jax.experimental.pallas import tpu_sc as plsc

IMG = 224
PIX = IMG * IMG
NB = 64
NCH = 3
NWORK = 32
TASKS = NB * NCH
TPW = TASKS // NWORK
ROWS_PER_CHUNK = 8
P = IMG * ROWS_PER_CHUNK
NCHUNK = PIX // P
GPR = IMG // 16
LO = np.float32(0.001)
HI = np.float32(IMG - 1.001)


def _frac_parts(q, dval):
    fi = q.astype(jnp.int32)
    ff = fi.astype(jnp.float32)
    fr = q - ff
    wf = 1.0 - fr
    nz = fr != 0.0
    wc = jnp.where(nz, fr, jnp.float32(1.0))
    d = jnp.where(nz, jnp.int32(dval), jnp.int32(0))
    return fi, wf, wc, d


def _sample(plane, q0, q1, mask):
    f0, wf0, wc0, d0 = _frac_parts(q0, IMG)
    f1, wf1, wc1, d1 = _frac_parts(q1, 1)
    iff = f1 + IMG * f0
    icf = iff + d0
    ifc = iff + d1
    icc = icf + d1
    wf0m = wf0 * mask
    wc0m = wc0 * mask
    g_ff = plsc.load_gather(plane, [iff])
    g_cf = plsc.load_gather(plane, [icf])
    g_fc = plsc.load_gather(plane, [ifc])
    g_cc = plsc.load_gather(plane, [icc])
    return ((wf0m * wf1) * g_ff + (wc0m * wf1) * g_cf
            + (wf0m * wc1) * g_fc + (wc0m * wc1) * g_cc)


def _body(im1, im2, C, M1, M2, out,
          plane1, plane2,
          c0b0, c1b0, m1b0, m2b0, c0b1, c1b1, m1b1, m2b1,
          outb0, outb1, colcb, rowcb,
          sem_pl, sem_in0, sem_in1, sem_out0, sem_out1):
    wid = lax.axis_index("s") * 2 + lax.axis_index("c")
    iotaf = lax.iota(jnp.int32, 16).astype(jnp.float32)
    zeros16 = jnp.zeros((16,), jnp.float32)

    inbufs = ((c0b0, c1b0, m1b0, m2b0), (c0b1, c1b1, m1b1, m2b1))
    outbufs = (outb0, outb1)
    sem_in = (sem_in0, sem_in1)
    sem_out = (sem_out0, sem_out1)

    def init_row(r, carry):
        rf = r.astype(jnp.float32) + zeros16

        def init_grp(g, carry):
            j = (r * GPR + g) * 16
            colcb[pl.ds(j, 16)] = iotaf + (g * 16).astype(jnp.float32)
            rowcb[pl.ds(j, 16)] = rf
            return carry

        return lax.fori_loop(0, GPR, init_grp, carry)

    lax.fori_loop(0, ROWS_PER_CHUNK, init_row, 0)

    def do_task(t, carry):
        task = wid * TPW + t
        n = task // NCH
        ch = task - n * NCH

        def fire_in(k, b):
            off = k * P
            c0b, c1b, m1b, m2b = inbufs[b]
            pltpu.async_copy(C.at[n, 0, pl.ds(off, P)], c0b, sem_in[b])
            pltpu.async_copy(C.at[n, 1, pl.ds(off, P)], c1b, sem_in[b])
            pltpu.async_copy(M1.at[n, 0, pl.ds(off, P)], m1b, sem_in[b])
            pltpu.async_copy(M2.at[n, 0, pl.ds(off, P)], m2b, sem_in[b])

        def drain_in(b):
            for ref in inbufs[b]:
                pltpu.make_async_copy(C.at[n, 0, pl.ds(0, P)], ref,
                                      sem_in[b]).wait()

        def wait_out(b):
            pltpu.make_async_copy(outbufs[b],
                                  out.at[n, ch, pl.ds(0, P)],
                                  sem_out[b]).wait()

        cp1 = pltpu.async_copy(im1.at[n, ch], plane1, sem_pl)
        cp2 = pltpu.async_copy(im2.at[n, ch], plane2, sem_pl)
        fire_in(0, 0)
        cp1.wait()
        cp2.wait()

        def do_pair(kk, carry):
            for b in range(2):
                k = kk * 2 + b
                if b == 0:
                    fire_in(k + 1, 1 - b)
                else:
                    @pl.when(kk < (NCHUNK // 2 - 1))
                    def _():
                        fire_in(k + 1, 1 - b)
                drain_in(b)

                @pl.when(kk >= 1)
                def _():
                    wait_out(b)

                c0b, c1b, m1b, m2b = inbufs[b]
                outb = outbufs[b]
                rowbase = (k * ROWS_PER_CHUNK).astype(jnp.float32) + zeros16
                off = k * P

                @plsc.parallel_loop(0, P, 16, unroll=2)
                def _(j):
                    colf = colcb[pl.ds(j, 16)]
                    rowf = rowbase + rowcb[pl.ds(j, 16)]
                    c0 = c0b[pl.ds(j, 16)]
                    c1 = c1b[pl.ds(j, 16)]
                    qa0 = jnp.clip(rowf + c0, LO, HI)
                    qa1 = jnp.clip(colf + c1, LO, HI)
                    qb0 = jnp.clip(rowf - c0, LO, HI)
                    qb1 = jnp.clip(colf - c1, LO, HI)
                    a = _sample(plane1, qa0, qa1, m1b[pl.ds(j, 16)])
                    b_ = _sample(plane2, qb0, qb1, m2b[pl.ds(j, 16)])
                    outb[pl.ds(j, 16)] = a + b_

                pltpu.async_copy(outb, out.at[n, ch, pl.ds(off, P)],
                                 sem_out[b])
            return carry

        lax.fori_loop(0, NCHUNK // 2, do_pair, 0)
        wait_out(0)
        wait_out(1)
        return carry

    lax.fori_loop(0, TPW, do_task, 0)


@jax.jit
def kernel(im1, im2, C, M1, M2):
    im1f = im1.reshape(NB, NCH, PIX)
    im2f = im2.reshape(NB, NCH, PIX)
    Cf = C.reshape(NB, 2, PIX)
    M1f = M1.reshape(NB, 1, PIX)
    M2f = M2.reshape(NB, 1, PIX)
    mesh = plsc.VectorSubcoreMesh(core_axis_name="c", subcore_axis_name="s")
    warp = pl.kernel(
        _body,
        out_type=jax.ShapeDtypeStruct((NB, NCH, PIX), jnp.float32),
        mesh=mesh,
        compiler_params=pltpu.CompilerParams(needs_layout_passes=False),
        scratch_types=[
            pltpu.VMEM((PIX,), jnp.float32),
            pltpu.VMEM((PIX,), jnp.float32),
        ] + [pltpu.VMEM((P,), jnp.float32)] * 12 + [
            pltpu.SemaphoreType.DMA,
            pltpu.SemaphoreType.DMA,
            pltpu.SemaphoreType.DMA,
            pltpu.SemaphoreType.DMA,
            pltpu.SemaphoreType.DMA,
        ],
    )
    out = warp(im1f, im2f, Cf, M1f, M2f)
    return out.reshape(NB, NCH, IMG, IMG)


# speedup vs baseline: 12.5845x; 1.5022x over previous
import functools

import numpy as np
import jax
import jax.numpy as jnp
from jax import lax
from jax.experimental import pallas as pl
from jax.experimental.pallas import tpu as pltpu
from jax.experimental.pallas import tpu_sc as plsc

IMG = 224
PIX = IMG * IMG
NB = 64
NCH = 3
NWORK = 32
TASKS = NB * NCH
TPW = TASKS // NWORK
ROWS_PER_CHUNK = 8
P = IMG * ROWS_PER_CHUNK
NCHUNK = PIX // P
LO = np.float32(0.001)
HI = np.float32(IMG - 1.001)
D0BIT = np.int32((224 << 24) - (1 << 32))
D1BIT = np.int32(1 << 16)


def _pack_bf16_pair(lo, hi):
    ul = lax.bitcast_convert_type(lo, jnp.uint32)
    uh = lax.bitcast_convert_type(hi, jnp.uint32)
    rl = (ul + jnp.uint32(0x7FFF) + ((ul >> 16) & 1)) >> 16
    rh = (uh + jnp.uint32(0x7FFF) + ((uh >> 16) & 1)) & jnp.uint32(0xFFFF0000)
    return lax.bitcast_convert_type(rl | rh, jnp.int32)


def _prep_side(q0, q1, m):
    q0 = jnp.clip(q0, LO, HI)
    q1 = jnp.clip(q1, LO, HI)
    f0 = q0.astype(jnp.int32)
    fr0 = q0 - f0.astype(jnp.float32)
    wf0 = 1.0 - fr0
    nz0 = fr0 != 0.0
    wc0 = jnp.where(nz0, fr0, jnp.float32(1.0))
    f1 = q1.astype(jnp.int32)
    fr1 = q1 - f1.astype(jnp.float32)
    wf1 = 1.0 - fr1
    nz1 = fr1 != 0.0
    wc1 = jnp.where(nz1, fr1, jnp.float32(1.0))
    K = ((f1 + IMG * f0)
         | jnp.where(nz1, D1BIT, jnp.int32(0))
         | jnp.where(nz0, D0BIT, jnp.int32(0)))
    A = _pack_bf16_pair(wf0 * m, wc0 * m)
    B = _pack_bf16_pair(wf1, wc1)
    return K, A, B


_BBATCH = 8
_BPIX = PIX // 7


def _prep_body(rowr, colr, c0r, c1r, m1r, m2r,
               kar, aar, bar, kbr, abr, bbr):
    row = rowr[...][None, :]
    col = colr[...][None, :]
    c0 = c0r[...]
    c1 = c1r[...]
    ka, aa, ba = _prep_side(row + c0, col + c1, m1r[...])
    kb, ab, bb = _prep_side(row - c0, col - c1, m2r[...])
    kar[...] = ka
    aar[...] = aa
    bar[...] = ba
    kbr[...] = kb
    abr[...] = ab
    bbr[...] = bb


def _make_prep():
    vspec = pl.BlockSpec((_BPIX,), lambda n, p: (p,))
    bspec = pl.BlockSpec((_BBATCH, _BPIX), lambda n, p: (n, p))
    ospec = jax.ShapeDtypeStruct((NB, PIX), jnp.int32)
    return pl.pallas_call(
        _prep_body,
        grid=(NB // _BBATCH, PIX // _BPIX),
        in_specs=[vspec, vspec, bspec, bspec, bspec, bspec],
        out_specs=[bspec] * 6,
        out_shape=[ospec] * 6,
    )


def _sample(plane, K, A, B):
    iff = K & 0xFFFF
    d1 = (K >> 16) & 1
    d0s = lax.shift_right_logical(K, 24)
    icf = iff + d0s
    ifc = iff + d1
    icc = icf + d1
    wf0m = plsc.bitcast(A << 16, jnp.float32)
    wc0m = plsc.bitcast(A, jnp.float32)
    wf1 = plsc.bitcast(B << 16, jnp.float32)
    wc1 = plsc.bitcast(B, jnp.float32)
    g_ff = plsc.load_gather(plane, [iff])
    g_cf = plsc.load_gather(plane, [icf])
    g_fc = plsc.load_gather(plane, [ifc])
    g_cc = plsc.load_gather(plane, [icc])
    return (wf1 * (wf0m * g_ff + wc0m * g_cf)
            + wc1 * (wf0m * g_fc + wc0m * g_cc))


def _body(im1, im2, Ka, Aa, Ba, Kb, Ab, Bb, out,
          plane1, plane2,
          ka0, aa0, ba0, kb0, ab0, bb0,
          ka1, aa1, ba1, kb1, ab1, bb1,
          outb0, outb1,
          sem_pl, sem_in0, sem_in1, sem_out0, sem_out1):
    wid = lax.axis_index("s") * 2 + lax.axis_index("c")

    srcs = (Ka, Aa, Ba, Kb, Ab, Bb)
    inbufs = ((ka0, aa0, ba0, kb0, ab0, bb0), (ka1, aa1, ba1, kb1, ab1, bb1))
    outbufs = (outb0, outb1)
    sem_in = (sem_in0, sem_in1)
    sem_out = (sem_out0, sem_out1)

    def do_task(t, carry):
        task = wid * TPW + t
        n = task // NCH

        def fire_in(k, b):
            off = k * P
            for src, ref in zip(srcs, inbufs[b]):
                pltpu.async_copy(src.at[n, pl.ds(off, P)], ref, sem_in[b])

        def drain_in(b):
            for ref in inbufs[b]:
                pltpu.make_async_copy(Ka.at[n, pl.ds(0, P)], ref,
                                      sem_in[b]).wait()

        def wait_out(b):
            pltpu.make_async_copy(outbufs[b],
                                  out.at[task, pl.ds(0, P)],
                                  sem_out[b]).wait()

        cp1 = pltpu.async_copy(im1.at[task], plane1, sem_pl)
        cp2 = pltpu.async_copy(im2.at[task], plane2, sem_pl)
        fire_in(0, 0)
        cp1.wait()
        cp2.wait()

        def do_pair(kk, carry):
            for b in range(2):
                k = kk * 2 + b
                if b == 0:
                    fire_in(k + 1, 1 - b)
                else:
                    @pl.when(kk < (NCHUNK // 2 - 1))
                    def _():
                        fire_in(k + 1, 1 - b)
                drain_in(b)

                @pl.when(kk >= 1)
                def _():
                    wait_out(b)

                kab, aab, bab, kbb, abb, bbb = inbufs[b]
                outb = outbufs[b]
                off = k * P

                @plsc.parallel_loop(0, P, 16, unroll=2)
                def _(j):
                    a = _sample(plane1, kab[pl.ds(j, 16)],
                                aab[pl.ds(j, 16)], bab[pl.ds(j, 16)])
                    b_ = _sample(plane2, kbb[pl.ds(j, 16)],
                                 abb[pl.ds(j, 16)], bbb[pl.ds(j, 16)])
                    outb[pl.ds(j, 16)] = a + b_

                pltpu.async_copy(outb, out.at[task, pl.ds(off, P)],
                                 sem_out[b])
            return carry

        lax.fori_loop(0, NCHUNK // 2, do_pair, 0)
        wait_out(0)
        wait_out(1)
        return carry

    lax.fori_loop(0, TPW, do_task, 0)


def _make_warp():
    mesh = plsc.VectorSubcoreMesh(core_axis_name="c", subcore_axis_name="s")
    return pl.kernel(
        _body,
        out_type=jax.ShapeDtypeStruct((NB * NCH, PIX), jnp.float32),
        mesh=mesh,
        compiler_params=pltpu.CompilerParams(needs_layout_passes=False),
        scratch_types=[
            pltpu.VMEM((PIX,), jnp.float32),
            pltpu.VMEM((PIX,), jnp.float32),
        ] + [pltpu.VMEM((P,), jnp.int32)] * 12 + [
            pltpu.VMEM((P,), jnp.float32),
            pltpu.VMEM((P,), jnp.float32),
            pltpu.SemaphoreType.DMA,
            pltpu.SemaphoreType.DMA,
            pltpu.SemaphoreType.DMA,
            pltpu.SemaphoreType.DMA,
            pltpu.SemaphoreType.DMA,
        ],
    )


@jax.jit
def kernel(im1, im2, C, M1, M2):
    c0f = C[:, 0].reshape(NB, PIX)
    c1f = C[:, 1].reshape(NB, PIX)
    M1f = M1.reshape(NB, PIX)
    M2f = M2.reshape(NB, PIX)
    i = lax.iota(jnp.int32, PIX)
    rowi = i // IMG
    rowf = rowi.astype(jnp.float32)
    colf = (i - rowi * IMG).astype(jnp.float32)
    Ka, Aa, Ba, Kb, Ab, Bb = _make_prep()(rowf, colf, c0f, c1f, M1f, M2f)
    im1f = im1.reshape(NB * NCH, PIX)
    im2f = im2.reshape(NB * NCH, PIX)
    out = _make_warp()(im1f, im2f, Ka, Aa, Ba, Kb, Ab, Bb)
    return out.reshape(NB, NCH, IMG, IMG)
